# baseline (speedup 1.0000x reference)
import functools

import jax
import jax.numpy as jnp
from jax import lax
from jax.experimental import pallas as pl
from jax.experimental.pallas import tpu as pltpu
from jax.experimental.pallas import tpu_sc as plsc

_VOCAB = 100000
_D = 128
_B = 4096
_C = 200
_NW = 32
_GRP = 40
_GPS = 5
_SUP = _GPS * _GRP
_NSLOT = 4
_ROWS = _B * _C
_ROWS_PER_TILE = _ROWS // _NW
_NSUPER = _ROWS_PER_TILE // _SUP


def _embed_body(tok_hbm, tab_hbm, out_hbm,
                idx0, idx1, idx2, idx3, buf0, buf1, buf2, buf3,
                gsem0, gsem1, gsem2, gsem3,
                osem0, osem1, osem2, osem3,
                isem0, isem1, isem2, isem3):
    nc = 2
    wid = lax.axis_index("s") * nc + lax.axis_index("c")
    r0 = wid * _ROWS_PER_TILE
    lane = lax.iota(jnp.int32, 16)
    is_last = lane == 15
    idx = (idx0, idx1, idx2, idx3)
    buf = (buf0, buf1, buf2, buf3)
    gsem = (gsem0, gsem1, gsem2, gsem3)
    osem = (osem0, osem1, osem2, osem3)
    isem = (isem0, isem1, isem2, isem3)

    def fire_idx(k, s):
        row0 = r0 + k * _SUP
        pltpu.async_copy(tok_hbm.at[pl.ds(row0, _SUP)], idx[s], isem[s])

    def wait_idx(s):
        pltpu.make_async_copy(
            tok_hbm.at[pl.ds(0, _SUP)], idx[s], isem[s]
        ).wait()

    def run_gathers(k, s):
        descs = [
            pltpu.async_copy(
                tab_hbm.at[idx[s].at[pl.ds(j * _GRP, _GRP)]],
                buf[s].at[pl.ds(j * _GRP, _GRP)],
                gsem[s],
            )
            for j in range(_GPS)
        ]
        for d in descs:
            d.wait()

    def blend_pos(s):
        for r in range(_SUP):
            p = jnp.float32((r % _C) / (_C - 1) * 4.0 - 2.0)
            span = buf[s][r, pl.ds(_D - 16, 16)]
            buf[s][r, pl.ds(_D - 16, 16)] = jnp.where(is_last, p, span)

    def fire_out(k, s):
        row0 = r0 + k * _SUP
        pltpu.async_copy(buf[s], out_hbm.at[pl.ds(row0, _SUP)], osem[s])

    def wait_out(s):
        pltpu.make_async_copy(
            buf[s], out_hbm.at[pl.ds(0, _SUP)], osem[s]
        ).wait()

    def stage(k, s, first):
        if not first:
            wait_out(s)
        wait_idx(s)
        run_gathers(k, s)
        blend_pos(s)
        fire_out(k, s)

    for s in range(_NSLOT):
        fire_idx(s, s)
    for s in range(_NSLOT):
        stage(s, s, True)
        fire_idx(s + _NSLOT, s)

    @pl.loop(1, _NSUPER // _NSLOT - 1)
    def _(it):
        for s in range(_NSLOT):
            k = it * _NSLOT + s
            stage(k, s, False)
            fire_idx(k + _NSLOT, s)

    for s in range(_NSLOT):
        stage(_NSUPER - _NSLOT + s, s, False)
    for s in range(_NSLOT):
        wait_out(s)


_PADB = 2000


def _pad_body(tab_ref, out_ref):
    i = pl.program_id(0)
    out_ref[:, : _D - 2] = tab_ref[...]
    v = (i * _PADB + lax.broadcasted_iota(jnp.int32, (_PADB, 1), 0))
    v = v.astype(jnp.float32)
    out_ref[:, _D - 2 : _D - 1] = v / (_VOCAB - 1) * 4.0 - 2.0
    out_ref[:, _D - 1 : _D] = jnp.zeros((_PADB, 1), jnp.float32)


def _pad_table(table):
    return pl.pallas_call(
        _pad_body,
        grid=(_VOCAB // _PADB,),
        in_specs=[pl.BlockSpec((_PADB, _D - 2), lambda i: (i, 0))],
        out_specs=pl.BlockSpec((_PADB, _D), lambda i: (i, 0)),
        out_shape=jax.ShapeDtypeStruct((_VOCAB, _D), jnp.float32),
    )(table)


@jax.jit
def kernel(tokens, table):
    tab128 = _pad_table(table)
    tok_flat = tokens.reshape(_ROWS)

    run = pl.kernel(
        _embed_body,
        out_type=jax.ShapeDtypeStruct((_ROWS, _D), jnp.float32),
        mesh=plsc.VectorSubcoreMesh(core_axis_name="c", subcore_axis_name="s"),
        scratch_types=(
            [pltpu.VMEM((_SUP,), jnp.int32)] * _NSLOT
            + [pltpu.VMEM((_SUP, _D), jnp.float32)] * _NSLOT
            + [pltpu.SemaphoreType.DMA] * (3 * _NSLOT)
        ),
    )
    out = run(tok_flat, tab128)
    return out.reshape(_B, _C, _D)


# speedup vs baseline: 1.0534x; 1.0534x over previous
import functools

import jax
import jax.numpy as jnp
from jax import lax
from jax.experimental import pallas as pl
from jax.experimental.pallas import tpu as pltpu
from jax.experimental.pallas import tpu_sc as plsc

_VOCAB = 100000
_D = 128
_B = 4096
_C = 200
_NW = 32
_GRP = 80
_GPS = 5
_SUP = _GPS * _GRP
_NSLOT = 2
_ROWS = _B * _C
_ROWS_PER_TILE = _ROWS // _NW
_NSUPER = _ROWS_PER_TILE // _SUP


def _embed_body(tok_hbm, tab_hbm, out_hbm,
                idx0, idx1, buf0, buf1,
                gsem0, gsem1, osem0, osem1, isem0, isem1):
    nc = 2
    wid = lax.axis_index("s") * nc + lax.axis_index("c")
    r0 = wid * _ROWS_PER_TILE
    lane = lax.iota(jnp.int32, 16)
    is_last = lane == 15
    idx = (idx0, idx1)
    buf = (buf0, buf1)
    gsem = (gsem0, gsem1)
    osem = (osem0, osem1)
    isem = (isem0, isem1)

    def fire_idx(k, s):
        row0 = r0 + jnp.minimum(k, _NSUPER - 1) * _SUP
        pltpu.async_copy(tok_hbm.at[pl.ds(row0, _SUP)], idx[s], isem[s])

    def wait_idx(s):
        pltpu.make_async_copy(
            tok_hbm.at[pl.ds(0, _SUP)], idx[s], isem[s]
        ).wait()

    def run_gathers(k, s):
        descs = [
            pltpu.async_copy(
                tab_hbm.at[idx[s].at[pl.ds(j * _GRP, _GRP)]],
                buf[s].at[pl.ds(j * _GRP, _GRP)],
                gsem[s],
            )
            for j in range(_GPS)
        ]
        for d in descs:
            d.wait()
        fire_idx(k + _NSLOT, s)

    def blend_pos(s):
        for r in range(_SUP):
            p = jnp.float32((r % _C) / (_C - 1) * 4.0 - 2.0)
            span = buf[s][r, pl.ds(_D - 16, 16)]
            buf[s][r, pl.ds(_D - 16, 16)] = jnp.where(is_last, p, span)

    def fire_out(k, s):
        row0 = r0 + k * _SUP
        pltpu.async_copy(buf[s], out_hbm.at[pl.ds(row0, _SUP)], osem[s])

    def wait_out(s):
        pltpu.make_async_copy(
            buf[s], out_hbm.at[pl.ds(0, _SUP)], osem[s]
        ).wait()

    def stage(k, s, first):
        if not first:
            wait_out(s)
        wait_idx(s)
        run_gathers(k, s)
        blend_pos(s)
        fire_out(k, s)

    for s in range(_NSLOT):
        fire_idx(s, s)
    for s in range(_NSLOT):
        stage(s, s, True)

    @pl.loop(1, _NSUPER // _NSLOT)
    def _(it):
        for s in range(_NSLOT):
            stage(it * _NSLOT + s, s, False)

    for s in range(_NSLOT):
        wait_idx(s)
        wait_out(s)


_PADB = 2000


def _pad_body(tab_ref, out_ref):
    i = pl.program_id(0)
    out_ref[:, : _D - 2] = tab_ref[...]
    v = (i * _PADB + lax.broadcasted_iota(jnp.int32, (_PADB, 1), 0))
    v = v.astype(jnp.float32)
    out_ref[:, _D - 2 : _D - 1] = v / (_VOCAB - 1) * 4.0 - 2.0
    out_ref[:, _D - 1 : _D] = jnp.zeros((_PADB, 1), jnp.float32)


def _pad_table(table):
    return pl.pallas_call(
        _pad_body,
        grid=(_VOCAB // _PADB,),
        in_specs=[pl.BlockSpec((_PADB, _D - 2), lambda i: (i, 0))],
        out_specs=pl.BlockSpec((_PADB, _D), lambda i: (i, 0)),
        out_shape=jax.ShapeDtypeStruct((_VOCAB, _D), jnp.float32),
    )(table)


@jax.jit
def kernel(tokens, table):
    tab128 = _pad_table(table)
    tok_flat = tokens.reshape(_ROWS)

    run = pl.kernel(
        _embed_body,
        out_type=jax.ShapeDtypeStruct((_ROWS, _D), jnp.float32),
        mesh=plsc.VectorSubcoreMesh(core_axis_name="c", subcore_axis_name="s"),
        scratch_types=(
            [pltpu.VMEM((_SUP,), jnp.int32)] * _NSLOT
            + [pltpu.VMEM((_SUP, _D), jnp.float32)] * _NSLOT
            + [pltpu.SemaphoreType.DMA] * (3 * _NSLOT)
        ),
    )
    out = run(tok_flat, tab128)
    return out.reshape(_B, _C, _D)


# speedup vs baseline: 1.0977x; 1.0420x over previous
import functools

import jax
import jax.numpy as jnp
from jax import lax
from jax.experimental import pallas as pl
from jax.experimental.pallas import tpu as pltpu
from jax.experimental.pallas import tpu_sc as plsc

_VOCAB = 100000
_D = 128
_B = 4096
_C = 200
_NW = 32
_GRP = 80
_GPS = 5
_SUP = _GPS * _GRP
_NSLOT = 2
_ROWS = _B * _C
_ROWS_PER_TILE = _ROWS // _NW
_NSUPER = _ROWS_PER_TILE // _SUP


def _embed_body(tok_hbm, tab_hbm, out_hbm,
                idx0, idx1, buf0, buf1,
                gsem0, gsem1, osem0, osem1, isem0, isem1):
    nc = 2
    wid = lax.axis_index("s") * nc + lax.axis_index("c")
    r0 = wid * _ROWS_PER_TILE
    lane = lax.iota(jnp.int32, 16)
    is_last = lane == 15
    idx = (idx0, idx1)
    buf = (buf0, buf1)
    gsem = (gsem0, gsem1)
    osem = (osem0, osem1)
    isem = (isem0, isem1)

    def fire_idx(k, s):
        row0 = r0 + jnp.minimum(k, _NSUPER - 1) * _SUP
        pltpu.async_copy(tok_hbm.at[pl.ds(row0, _SUP)], idx[s], isem[s])

    def wait_idx(s):
        pltpu.make_async_copy(
            tok_hbm.at[pl.ds(0, _SUP)], idx[s], isem[s]
        ).wait()

    def run_gathers(k, s):
        descs = [
            pltpu.async_copy(
                tab_hbm.at[idx[s].at[pl.ds(j * _GRP, _GRP)]],
                buf[s].at[pl.ds(j * _GRP, _GRP)],
                gsem[s],
            )
            for j in range(_GPS)
        ]
        for d in descs:
            d.wait()
        fire_idx(k + _NSLOT, s)

    def blend_pos(s):
        for r in range(_SUP):
            p = jnp.float32((r % _C) / (_C - 1) * 4.0 - 2.0)
            plsc.addupdate(
                buf[s].at[r, pl.ds(_D - 16, 16)],
                jnp.where(is_last, p, 0.0),
            )

    def fire_out(k, s):
        row0 = r0 + k * _SUP
        pltpu.async_copy(buf[s], out_hbm.at[pl.ds(row0, _SUP)], osem[s])

    def wait_out(s):
        pltpu.make_async_copy(
            buf[s], out_hbm.at[pl.ds(0, _SUP)], osem[s]
        ).wait()

    def stage(k, s, first):
        if not first:
            wait_out(s)
        wait_idx(s)
        run_gathers(k, s)
        blend_pos(s)
        fire_out(k, s)

    for s in range(_NSLOT):
        fire_idx(s, s)
    for s in range(_NSLOT):
        stage(s, s, True)

    @pl.loop(1, _NSUPER // _NSLOT)
    def _(it):
        for s in range(_NSLOT):
            stage(it * _NSLOT + s, s, False)

    for s in range(_NSLOT):
        wait_idx(s)
        wait_out(s)


_PADB = 10000


def _pad_body(tab_ref, out_ref):
    i = pl.program_id(0)
    out_ref[:, : _D - 2] = tab_ref[...]
    v = (i * _PADB + lax.broadcasted_iota(jnp.int32, (_PADB, 1), 0))
    v = v.astype(jnp.float32)
    out_ref[:, _D - 2 : _D - 1] = v / (_VOCAB - 1) * 4.0 - 2.0
    out_ref[:, _D - 1 : _D] = jnp.zeros((_PADB, 1), jnp.float32)


def _pad_table(table):
    return pl.pallas_call(
        _pad_body,
        grid=(_VOCAB // _PADB,),
        in_specs=[pl.BlockSpec((_PADB, _D - 2), lambda i: (i, 0))],
        out_specs=pl.BlockSpec((_PADB, _D), lambda i: (i, 0)),
        out_shape=jax.ShapeDtypeStruct((_VOCAB, _D), jnp.float32),
    )(table)


@jax.jit
def kernel(tokens, table):
    tab128 = _pad_table(table)
    tok_flat = tokens.reshape(_ROWS)

    run = pl.kernel(
        _embed_body,
        out_type=jax.ShapeDtypeStruct((_ROWS, _D), jnp.float32),
        mesh=plsc.VectorSubcoreMesh(core_axis_name="c", subcore_axis_name="s"),
        scratch_types=(
            [pltpu.VMEM((_SUP,), jnp.int32)] * _NSLOT
            + [pltpu.VMEM((_SUP, _D), jnp.float32)] * _NSLOT
            + [pltpu.SemaphoreType.DMA] * (3 * _NSLOT)
        ),
    )
    out = run(tok_flat, tab128)
    return out.reshape(_B, _C, _D)


# speedup vs baseline: 1.1125x; 1.0135x over previous
import functools

import jax
import jax.numpy as jnp
from jax import lax
from jax.experimental import pallas as pl
from jax.experimental.pallas import tpu as pltpu
from jax.experimental.pallas import tpu_sc as plsc

_VOCAB = 100000
_D = 128
_B = 4096
_C = 200
_NW = 32
_GRP = 80
_GPS = 5
_SUP = _GPS * _GRP
_NSLOT = 2
_ROWS = _B * _C
_ROWS_PER_TILE = _ROWS // _NW
_NSUPER = _ROWS_PER_TILE // _SUP


def _embed_body(tok_hbm, tab_hbm, out_hbm,
                idx0, idx1, buf0, buf1,
                g00, g01, g02, g03, g04, g10, g11, g12, g13, g14,
                osem0, osem1, isem0, isem1):
    nc = 2
    wid = lax.axis_index("s") * nc + lax.axis_index("c")
    r0 = wid * _ROWS_PER_TILE
    lane = lax.iota(jnp.int32, 16)
    is_last = lane == 15
    idx = (idx0, idx1)
    buf = (buf0, buf1)
    gsem = ((g00, g01, g02, g03, g04), (g10, g11, g12, g13, g14))
    osem = (osem0, osem1)
    isem = (isem0, isem1)

    def fire_idx(k, s):
        row0 = r0 + jnp.minimum(k, _NSUPER - 1) * _SUP
        pltpu.async_copy(tok_hbm.at[pl.ds(row0, _SUP)], idx[s], isem[s])

    def wait_idx(s):
        pltpu.make_async_copy(
            tok_hbm.at[pl.ds(0, _SUP)], idx[s], isem[s]
        ).wait()

    def blend_pos(s, j):
        for g in range(_GRP):
            r = j * _GRP + g
            p = jnp.float32((r % _C) / (_C - 1) * 4.0 - 2.0)
            plsc.addupdate(
                buf[s].at[r, pl.ds(_D - 16, 16)],
                jnp.where(is_last, p, 0.0),
            )

    def wait_out(s):
        for j in range(_GPS):
            pltpu.make_async_copy(
                buf[s].at[pl.ds(j * _GRP, _GRP)],
                out_hbm.at[pl.ds(0, _GRP)],
                osem[s],
            ).wait()

    def stage(k, s, first):
        if not first:
            wait_out(s)
        wait_idx(s)
        row0 = r0 + k * _SUP
        descs = [
            pltpu.async_copy(
                tab_hbm.at[idx[s].at[pl.ds(j * _GRP, _GRP)]],
                buf[s].at[pl.ds(j * _GRP, _GRP)],
                gsem[s][j],
            )
            for j in range(_GPS)
        ]
        for j in range(_GPS):
            descs[j].wait()
            if j == _GPS - 1:
                fire_idx(k + _NSLOT, s)
            blend_pos(s, j)
            pltpu.async_copy(
                buf[s].at[pl.ds(j * _GRP, _GRP)],
                out_hbm.at[pl.ds(row0 + j * _GRP, _GRP)],
                osem[s],
            )

    for s in range(_NSLOT):
        fire_idx(s, s)
    for s in range(_NSLOT):
        stage(s, s, True)

    @pl.loop(1, _NSUPER // _NSLOT)
    def _(it):
        for s in range(_NSLOT):
            stage(it * _NSLOT + s, s, False)

    for s in range(_NSLOT):
        wait_idx(s)
        wait_out(s)


_PADB = 10000


def _pad_body(tab_ref, out_ref):
    i = pl.program_id(0)
    out_ref[:, : _D - 2] = tab_ref[...]
    v = (i * _PADB + lax.broadcasted_iota(jnp.int32, (_PADB, 1), 0))
    v = v.astype(jnp.float32)
    out_ref[:, _D - 2 : _D - 1] = v / (_VOCAB - 1) * 4.0 - 2.0
    out_ref[:, _D - 1 : _D] = jnp.zeros((_PADB, 1), jnp.float32)


def _pad_table(table):
    return pl.pallas_call(
        _pad_body,
        grid=(_VOCAB // _PADB,),
        in_specs=[pl.BlockSpec((_PADB, _D - 2), lambda i: (i, 0))],
        out_specs=pl.BlockSpec((_PADB, _D), lambda i: (i, 0)),
        out_shape=jax.ShapeDtypeStruct((_VOCAB, _D), jnp.float32),
    )(table)


@jax.jit
def kernel(tokens, table):
    tab128 = _pad_table(table)
    tok_flat = tokens.reshape(_ROWS)

    run = pl.kernel(
        _embed_body,
        out_type=jax.ShapeDtypeStruct((_ROWS, _D), jnp.float32),
        mesh=plsc.VectorSubcoreMesh(core_axis_name="c", subcore_axis_name="s"),
        scratch_types=(
            [pltpu.VMEM((_SUP,), jnp.int32)] * _NSLOT
            + [pltpu.VMEM((_SUP, _D), jnp.float32)] * _NSLOT
            + [pltpu.SemaphoreType.DMA] * (_GPS * _NSLOT + 2 * _NSLOT)
        ),
    )
    out = run(tok_flat, tab128)
    return out.reshape(_B, _C, _D)
